# baseline (speedup 1.0000x reference)
import jax
import jax.numpy as jnp
from jax import lax
from jax.experimental import pallas as pl
from jax.experimental.pallas import tpu as pltpu
from jax.experimental.pallas import tpu_sc as plsc

D = 256
K = 2048
B = 16
BH = 8
HW = 1024
NROWS = B * HW
COMMIT = 0.25

NC = 2
NS = 16
NW = NC * NS
DPW = D // NW
L = 16


def _tc_body(x_ref, emb_ref, cin_ref, lin_ref,
             idx_ref, cout_ref, lout_ref, loss_ref, perp_ref,
             counts_acc, loss_acc):
    b = pl.program_id(0)
    x = x_ref[0]
    emb = emb_ref[...]
    m2 = lax.dot_general(x, emb + emb, (((0,), (0,)), ((), ())),
                         preferred_element_type=jnp.float32)
    ones = jnp.ones((D, 1), jnp.float32)
    x2 = lax.dot_general(x * x, ones, (((0,), (0,)), ((), ())),
                         preferred_element_type=jnp.float32)
    e2 = jnp.sum(emb * emb, axis=0, keepdims=True)
    dist = (x2 + e2) - m2
    dmin = jnp.min(dist, axis=1)
    eqm = dist == dmin[:, None]
    colsf = lax.broadcasted_iota(jnp.int32, (HW, K), 1).astype(jnp.float32)
    sel = jnp.where(eqm, colsf, jnp.float32(K))
    idx = jnp.min(sel, axis=1).astype(jnp.int32)
    idx_ref[...] = idx.reshape(8, 128)

    cblock = jnp.sum(eqm.astype(jnp.float32), axis=0, keepdims=True)

    @pl.when(b == 0)
    def _():
        counts_acc[...] = cin_ref[...]
        loss_acc[0, 0] = lin_ref[0]

    counts_acc[...] += cblock
    loss_acc[0, 0] += jnp.sum(dmin)

    @pl.when(b == BH - 1)
    def _():
        cout_ref[...] = counts_acc[...]
        lout_ref[0] = loss_acc[0, 0]
        loss_ref[...] = jnp.full(
            (1, 1), loss_acc[0, 0] * ((1.0 + COMMIT) / (NROWS * D)), jnp.float32)
        p = counts_acc[...] * (1.0 / NROWS)
        perp_ref[...] = jnp.exp(
            -jnp.sum(p * jnp.log(p + 1e-10), keepdims=True))


@jax.jit
def _tc_call(x3, emb, cin, lin):
    return pl.pallas_call(
        _tc_body,
        grid=(BH,),
        in_specs=[
            pl.BlockSpec((1, D, HW), lambda b: (b, 0, 0)),
            pl.BlockSpec((D, K), lambda b: (0, 0)),
            pl.BlockSpec((1, K), lambda b: (0, 0)),
            pl.BlockSpec(memory_space=pltpu.SMEM),
        ],
        out_specs=[
            pl.BlockSpec((8, 128), lambda b: (b, 0)),
            pl.BlockSpec((1, K), lambda b: (0, 0)),
            pl.BlockSpec(memory_space=pltpu.SMEM),
            pl.BlockSpec((1, 1), lambda b: (0, 0)),
            pl.BlockSpec((1, 1), lambda b: (0, 0)),
        ],
        out_shape=[
            jax.ShapeDtypeStruct((BH * 8, 128), jnp.int32),
            jax.ShapeDtypeStruct((1, K), jnp.float32),
            jax.ShapeDtypeStruct((1,), jnp.float32),
            jax.ShapeDtypeStruct((1, 1), jnp.float32),
            jax.ShapeDtypeStruct((1, 1), jnp.float32),
        ],
        scratch_shapes=[
            pltpu.VMEM((1, K), jnp.float32),
            pltpu.SMEM((1, 1), jnp.float32),
        ],
        compiler_params=pltpu.CompilerParams(
            dimension_semantics=("arbitrary",),
        ),
    )(x3, emb, cin, lin)


def _sc_body(emb_hbm, idx_hbm, out_hbm, idx_v, rows_v, buf_v):
    wid = lax.axis_index("s") * NC + lax.axis_index("c")
    d0 = wid * DPW
    pltpu.sync_copy(idx_hbm, idx_v)
    pltpu.sync_copy(emb_hbm.at[pl.ds(d0, DPW)], rows_v)

    dvs = [jnp.full((L,), dl, jnp.int32) for dl in range(DPW)]
    GRP = 4

    def body_b(b, carry):
        b8 = b * 8
        for i in range(0, HW // L, GRP):
            ivs = [idx_v[b8 + (i + u) // 8, pl.ds(L * ((i + u) % 8), L)]
                   for u in range(GRP)]
            vals = [[plsc.load_gather(rows_v, [dvs[dl], ivs[u]])
                     for dl in range(DPW)] for u in range(GRP)]
            for u in range(GRP):
                for dl in range(DPW):
                    buf_v[dl, pl.ds(L * (i + u), L)] = vals[u][dl]
        pltpu.sync_copy(buf_v, out_hbm.at[b, pl.ds(d0, DPW)])
        return carry

    lax.fori_loop(0, BH, body_b, 0)


@jax.jit
def _sc_call(emb, idx2):
    mesh = plsc.VectorSubcoreMesh(core_axis_name="c", subcore_axis_name="s")
    f = pl.kernel(
        _sc_body,
        out_type=jax.ShapeDtypeStruct((BH, D, HW), jnp.float32),
        mesh=mesh,
        scratch_types=[
            pltpu.VMEM((BH * 8, 128), jnp.int32),
            pltpu.VMEM((DPW, K), jnp.float32),
            pltpu.VMEM((DPW, HW), jnp.float32),
        ],
        compiler_params=pltpu.CompilerParams(
            use_tc_tiling_on_sc=False, needs_layout_passes=False),
    )
    return f(emb, idx2)


def kernel(inputs, embedding):
    x3 = inputs.reshape(B, D, HW)
    zc = jnp.zeros((1, K), jnp.float32)
    zl = jnp.zeros((1,), jnp.float32)
    idxA, cA, lA, _, _ = _tc_call(x3[:BH], embedding, zc, zl)
    qA = _sc_call(embedding, idxA)
    idxB, _, _, loss, perp = _tc_call(x3[BH:], embedding, cA, lA)
    qB = _sc_call(embedding, idxB)
    quantized_st = jnp.concatenate(
        [qA.reshape(BH, D, 32, 32), qB.reshape(BH, D, 32, 32)], axis=0)
    idx_flat = jnp.concatenate(
        [idxA.reshape(BH * HW), idxB.reshape(BH * HW)])
    return (quantized_st, loss[0, 0], perp[0, 0], idx_flat)


# speedup vs baseline: 1.1212x; 1.1212x over previous
import jax
import jax.numpy as jnp
from jax import lax
from jax.experimental import pallas as pl
from jax.experimental.pallas import tpu as pltpu
from jax.experimental.pallas import tpu_sc as plsc

D = 256
K = 2048
B = 16
BH = 8
HW = 1024
NROWS = B * HW
COMMIT = 0.25

NC = 2
NS = 16
NW = NC * NS
DPW = D // NW
L = 16


def _tc_body(x_ref, emb_ref, cin_ref, lin_ref,
             idx_ref, cout_ref, lout_ref, loss_ref, perp_ref,
             counts_acc, loss_acc):
    b = pl.program_id(0)
    x = x_ref[0]
    emb = emb_ref[...]
    m2 = lax.dot_general(x, emb + emb, (((0,), (0,)), ((), ())),
                         preferred_element_type=jnp.float32)
    ones = jnp.ones((D, 1), jnp.float32)
    x2 = lax.dot_general(x * x, ones, (((0,), (0,)), ((), ())),
                         preferred_element_type=jnp.float32)
    e2 = jnp.sum(emb * emb, axis=0, keepdims=True)
    dist = (x2 + e2) - m2
    dmin = jnp.min(dist, axis=1)
    eqm = dist == dmin[:, None]
    colsf = lax.broadcasted_iota(jnp.int32, (HW, K), 1).astype(jnp.float32)
    sel = jnp.where(eqm, colsf, jnp.float32(K))
    idx = jnp.min(sel, axis=1).astype(jnp.int32)
    idx_ref[...] = idx.reshape(8, 128)

    cblock = jnp.sum(eqm.astype(jnp.float32), axis=0, keepdims=True)

    @pl.when(b == 0)
    def _():
        counts_acc[...] = cin_ref[...]
        loss_acc[0, 0] = lin_ref[0]

    counts_acc[...] += cblock
    loss_acc[0, 0] += jnp.sum(dmin)

    @pl.when(b == BH - 1)
    def _():
        cout_ref[...] = counts_acc[...]
        lout_ref[0] = loss_acc[0, 0]
        loss_ref[...] = jnp.full(
            (1, 1), loss_acc[0, 0] * ((1.0 + COMMIT) / (NROWS * D)), jnp.float32)
        p = counts_acc[...] * (1.0 / NROWS)
        perp_ref[...] = jnp.exp(
            -jnp.sum(p * jnp.log(p + 1e-10), keepdims=True))


import functools


@functools.partial(jax.jit, static_argnums=(4,))
def _tc_call(x3, emb, cin, lin, base):
    return pl.pallas_call(
        _tc_body,
        grid=(BH,),
        in_specs=[
            pl.BlockSpec((1, D, HW), lambda b: (b + base, 0, 0)),
            pl.BlockSpec((D, K), lambda b: (0, 0)),
            pl.BlockSpec((1, K), lambda b: (0, 0)),
            pl.BlockSpec(memory_space=pltpu.SMEM),
        ],
        out_specs=[
            pl.BlockSpec((8, 128), lambda b: (b, 0)),
            pl.BlockSpec((1, K), lambda b: (0, 0)),
            pl.BlockSpec(memory_space=pltpu.SMEM),
            pl.BlockSpec((1, 1), lambda b: (0, 0)),
            pl.BlockSpec((1, 1), lambda b: (0, 0)),
        ],
        out_shape=[
            jax.ShapeDtypeStruct((BH * 8, 128), jnp.int32),
            jax.ShapeDtypeStruct((1, K), jnp.float32),
            jax.ShapeDtypeStruct((1,), jnp.float32),
            jax.ShapeDtypeStruct((1, 1), jnp.float32),
            jax.ShapeDtypeStruct((1, 1), jnp.float32),
        ],
        scratch_shapes=[
            pltpu.VMEM((1, K), jnp.float32),
            pltpu.SMEM((1, 1), jnp.float32),
        ],
        compiler_params=pltpu.CompilerParams(
            dimension_semantics=("arbitrary",),
        ),
    )(x3, emb, cin, lin)


def _sc_body(emb_hbm, idx_hbm, out_hbm, idx_v, rows_v, buf_v):
    wid = lax.axis_index("s") * NC + lax.axis_index("c")
    d0 = wid * DPW
    pltpu.sync_copy(idx_hbm, idx_v)
    pltpu.sync_copy(emb_hbm.at[pl.ds(d0, DPW)], rows_v)

    dvs = [jnp.full((L,), dl, jnp.int32) for dl in range(DPW)]
    GRP = 4

    def body_b(b, carry):
        b8 = b * 8
        for i in range(0, HW // L, GRP):
            ivs = [idx_v[b8 + (i + u) // 8, pl.ds(L * ((i + u) % 8), L)]
                   for u in range(GRP)]
            vals = [[plsc.load_gather(rows_v, [dvs[dl], ivs[u]])
                     for dl in range(DPW)] for u in range(GRP)]
            for u in range(GRP):
                for dl in range(DPW):
                    buf_v[dl, pl.ds(L * (i + u), L)] = vals[u][dl]
        pltpu.sync_copy(buf_v, out_hbm.at[b, pl.ds(d0, DPW)])
        return carry

    lax.fori_loop(0, BH, body_b, 0)


@jax.jit
def _sc_call(emb, idx2):
    mesh = plsc.VectorSubcoreMesh(core_axis_name="c", subcore_axis_name="s")
    f = pl.kernel(
        _sc_body,
        out_type=jax.ShapeDtypeStruct((BH, D, HW), jnp.float32),
        mesh=mesh,
        scratch_types=[
            pltpu.VMEM((BH * 8, 128), jnp.int32),
            pltpu.VMEM((DPW, K), jnp.float32),
            pltpu.VMEM((DPW, HW), jnp.float32),
        ],
        compiler_params=pltpu.CompilerParams(
            use_tc_tiling_on_sc=False, needs_layout_passes=False),
    )
    return f(emb, idx2)


def kernel(inputs, embedding):
    x3 = inputs.reshape(B, D, HW)
    zc = jnp.zeros((1, K), jnp.float32)
    zl = jnp.zeros((1,), jnp.float32)
    idxA, cA, lA, _, _ = _tc_call(x3, embedding, zc, zl, 0)
    qA = _sc_call(embedding, idxA)
    idxB, _, _, loss, perp = _tc_call(x3, embedding, cA, lA, BH)
    qB = _sc_call(embedding, idxB)
    quantized_st = jnp.concatenate(
        [qA.reshape(BH, D, 32, 32), qB.reshape(BH, D, 32, 32)], axis=0)
    idx_flat = jnp.concatenate(
        [idxA.reshape(BH * HW), idxB.reshape(BH * HW)])
    return (quantized_st, loss[0, 0], perp[0, 0], idx_flat)


# speedup vs baseline: 1.1287x; 1.0067x over previous
import jax
import jax.numpy as jnp
from jax import lax
from jax.experimental import pallas as pl
from jax.experimental.pallas import tpu as pltpu
from jax.experimental.pallas import tpu_sc as plsc

D = 256
K = 2048
B = 16
BH = 8
HW = 1024
NROWS = B * HW
COMMIT = 0.25

NC = 2
NS = 16
NW = NC * NS
DPW = D // NW
L = 16


def _tc_body(x_ref, emb_ref, cin_ref, lin_ref,
             idx_ref, cout_ref, lout_ref, loss_ref, perp_ref,
             counts_acc, loss_acc):
    b = pl.program_id(0)
    x = x_ref[0]
    emb = emb_ref[...]
    m2 = lax.dot_general(x, emb + emb, (((0,), (0,)), ((), ())),
                         preferred_element_type=jnp.float32)
    ones = jnp.ones((D, 1), jnp.float32)
    x2 = lax.dot_general(x * x, ones, (((0,), (0,)), ((), ())),
                         preferred_element_type=jnp.float32)
    e2 = jnp.sum(emb * emb, axis=0, keepdims=True)
    dist = (x2 + e2) - m2
    dmin = jnp.min(dist, axis=1)
    eqm = dist == dmin[:, None]
    colsf = lax.broadcasted_iota(jnp.int32, (HW, K), 1).astype(jnp.float32)
    sel = jnp.where(eqm, colsf, jnp.float32(K))
    idx = jnp.min(sel, axis=1).astype(jnp.int32)
    idx_ref[...] = idx.reshape(8, 128)

    cblock = jnp.sum(eqm.astype(jnp.float32), axis=0, keepdims=True)

    @pl.when(b == 0)
    def _():
        counts_acc[...] = cin_ref[...]
        loss_acc[0, 0] = lin_ref[0]

    counts_acc[...] += cblock
    loss_acc[0, 0] += jnp.sum(dmin)

    @pl.when(b == BH - 1)
    def _():
        cout_ref[...] = counts_acc[...]
        lout_ref[0] = loss_acc[0, 0]
        loss_ref[...] = jnp.full(
            (1, 1), loss_acc[0, 0] * ((1.0 + COMMIT) / (NROWS * D)), jnp.float32)
        p = counts_acc[...] * (1.0 / NROWS)
        perp_ref[...] = jnp.exp(
            -jnp.sum(p * jnp.log(p + 1e-10), keepdims=True))


import functools


@functools.partial(jax.jit, static_argnums=(4,))
def _tc_call(x3, emb, cin, lin, base):
    return pl.pallas_call(
        _tc_body,
        grid=(BH,),
        in_specs=[
            pl.BlockSpec((1, D, HW), lambda b: (b + base, 0, 0)),
            pl.BlockSpec((D, K), lambda b: (0, 0)),
            pl.BlockSpec((1, K), lambda b: (0, 0)),
            pl.BlockSpec(memory_space=pltpu.SMEM),
        ],
        out_specs=[
            pl.BlockSpec((8, 128), lambda b: (b, 0)),
            pl.BlockSpec((1, K), lambda b: (0, 0)),
            pl.BlockSpec(memory_space=pltpu.SMEM),
            pl.BlockSpec((1, 1), lambda b: (0, 0)),
            pl.BlockSpec((1, 1), lambda b: (0, 0)),
        ],
        out_shape=[
            jax.ShapeDtypeStruct((BH * 8, 128), jnp.int32),
            jax.ShapeDtypeStruct((1, K), jnp.float32),
            jax.ShapeDtypeStruct((1,), jnp.float32),
            jax.ShapeDtypeStruct((1, 1), jnp.float32),
            jax.ShapeDtypeStruct((1, 1), jnp.float32),
        ],
        scratch_shapes=[
            pltpu.VMEM((1, K), jnp.float32),
            pltpu.SMEM((1, 1), jnp.float32),
        ],
        compiler_params=pltpu.CompilerParams(
            dimension_semantics=("arbitrary",),
        ),
    )(x3, emb, cin, lin)


def _sc_body(emb_hbm, idx_hbm, out_hbm, idx_v, rows_v, buf_v):
    wid = lax.axis_index("s") * NC + lax.axis_index("c")
    d0 = wid * DPW
    pltpu.sync_copy(idx_hbm, idx_v)
    pltpu.sync_copy(emb_hbm.at[pl.ds(d0, DPW)], rows_v)

    dvs = [jnp.full((L,), dl, jnp.int32) for dl in range(DPW)]
    GRP = 4

    def body_b(b, carry):
        b8 = b * 8
        for i in range(0, HW // L, GRP):
            ivs = [idx_v[b8 + (i + u) // 8, pl.ds(L * ((i + u) % 8), L)]
                   for u in range(GRP)]
            vals = [[plsc.load_gather(rows_v, [dvs[dl], ivs[u]])
                     for dl in range(DPW)] for u in range(GRP)]
            for u in range(GRP):
                for dl in range(DPW):
                    buf_v[dl, pl.ds(L * (i + u), L)] = vals[u][dl]
        pltpu.sync_copy(buf_v, out_hbm.at[b, pl.ds(d0, DPW)])
        return carry

    lax.fori_loop(0, BH, body_b, 0)


@jax.jit
def _sc_call(emb, idx2):
    mesh = plsc.VectorSubcoreMesh(core_axis_name="c", subcore_axis_name="s")
    f = pl.kernel(
        _sc_body,
        out_type=jax.ShapeDtypeStruct((BH, D, HW), jnp.float32),
        mesh=mesh,
        scratch_types=[
            pltpu.VMEM((BH * 8, 128), jnp.int32),
            pltpu.VMEM((DPW, K), jnp.float32),
            pltpu.VMEM((DPW, HW), jnp.float32),
        ],
        compiler_params=pltpu.CompilerParams(
            use_tc_tiling_on_sc=False, needs_layout_passes=False),
    )
    return f(emb, idx2)


def kernel(inputs, embedding):
    x3 = inputs.reshape(B, D, HW)
    zc = jnp.zeros((1, K), jnp.float32)
    zl = jnp.zeros((1,), jnp.float32)
    idxA, cA, lA, _, _ = _tc_call(x3, embedding, zc, zl, 0)
    qA = _sc_call(embedding, idxA)
    idxB, _, _, loss, perp = _tc_call(x3, embedding, cA, lA, BH)
    qB = _sc_call(embedding, idxB)
    quantized_st = jnp.concatenate([qA, qB], axis=0).reshape(B, D, 32, 32)
    idx_flat = jnp.concatenate(
        [idxA.reshape(BH * HW), idxB.reshape(BH * HW)])
    return (quantized_st, loss[0, 0], perp[0, 0], idx_flat)
